# Initial kernel scaffold; baseline (speedup 1.0000x reference)
import jax
import jax.numpy as jnp
from jax.experimental import pallas as pl


def kernel(x, W_gate, W_noise):
    raise NotImplementedError("write your pallas kernel here")


# speedup vs baseline: 3.8009x; 3.8009x over previous
import functools

import jax
import jax.numpy as jnp
from jax.experimental import pallas as pl
from jax.experimental.pallas import tpu as pltpu

_E = 8
_TOP_P = 0.7
_EPS = 1e-10


def _router_block(x_ref, wg_ref, w_ref, dec_ref, loss_ref, imp_acc, ent_acc):
    i = pl.program_id(0)
    nsteps = pl.num_programs(0)

    @pl.when(i == 0)
    def _init():
        for k in range(_E):
            imp_acc[0, k] = 0.0
        ent_acc[0, 0] = 0.0

    x = x_ref[...]
    wg = wg_ref[...]
    logits = jax.lax.dot_general(
        x, wg, (((1,), (1,)), ((), ())),
        preferred_element_type=jnp.float32)

    m = jnp.max(logits, axis=-1, keepdims=True)
    ex = jnp.exp(logits - m)
    p = ex / jnp.sum(ex, axis=-1, keepdims=True)

    col = jax.lax.broadcasted_iota(jnp.int32, p.shape, 1)
    s_cols = []
    r_cols = []
    for e in range(_E):
        pe = p[:, e:e + 1]
        higher = (p > pe) | ((p == pe) & (col < e))
        s_cols.append(jnp.sum(jnp.where(higher, p, 0.0), axis=-1, keepdims=True))
        r_cols.append(jnp.sum(higher.astype(jnp.int32), axis=-1, keepdims=True))
    s_above = jnp.concatenate(s_cols, axis=-1)
    rank = jnp.concatenate(r_cols, axis=-1)

    kept = s_above <= _TOP_P
    w_ref[...] = kept.astype(jnp.int8)
    cnt = jnp.sum(kept.astype(jnp.int32), axis=-1, keepdims=True)
    dec_ref[...] = (cnt > 1).astype(jnp.int32)

    contrib = jnp.where(kept, p, 0.0)
    for k in range(_E):
        part = jnp.sum(jnp.where(rank == k, contrib, 0.0))
        imp_acc[0, k] = imp_acc[0, k] + part
    ent_acc[0, 0] = ent_acc[0, 0] - jnp.sum(p * jnp.log(p + _EPS))

    @pl.when(i == nsteps - 1)
    def _fin():
        total = 0.0
        for k in range(_E):
            total = total + imp_acc[0, k]
        mean = total / _E
        var = 0.0
        for k in range(_E):
            d = imp_acc[0, k] - mean
            var = var + d * d
        var = var / (_E - 1)
        loss_imp = var / (mean * mean + _EPS)
        n_tokens = nsteps * x_ref.shape[0]
        loss_dyn = ent_acc[0, 0] / n_tokens
        loss_ref[0, 0] = loss_imp + 0.1 * loss_dyn


@functools.partial(jax.jit, static_argnames=())
def kernel(x, W_gate, W_noise):
    del W_noise
    b, n, d = x.shape
    e = W_gate.shape[0]
    bn = b * n
    t = 512
    grid = bn // t
    x_flat = x.reshape(bn, d)

    w_i8, dec, loss = pl.pallas_call(
        _router_block,
        grid=(grid,),
        in_specs=[
            pl.BlockSpec((t, d), lambda i: (i, 0)),
            pl.BlockSpec((e, d), lambda i: (0, 0)),
        ],
        out_specs=[
            pl.BlockSpec((t, e), lambda i: (i, 0)),
            pl.BlockSpec((t, 1), lambda i: (i, 0)),
            pl.BlockSpec(memory_space=pltpu.SMEM),
        ],
        out_shape=[
            jax.ShapeDtypeStruct((bn, e), jnp.int8),
            jax.ShapeDtypeStruct((bn, 1), jnp.int32),
            jax.ShapeDtypeStruct((1, 1), jnp.float32),
        ],
        scratch_shapes=[
            pltpu.SMEM((1, e), jnp.float32),
            pltpu.SMEM((1, 1), jnp.float32),
        ],
        compiler_params=pltpu.CompilerParams(
            dimension_semantics=("arbitrary",),
        ),
    )(x_flat, W_gate)

    expert_weights = w_i8.astype(jnp.bool_).reshape(b, n, e)
    expert_decisions = dec.reshape(b, n)
    gating_loss = loss.reshape(())
    return expert_weights, expert_decisions, gating_loss


# speedup vs baseline: 10.9188x; 2.8727x over previous
import functools

import jax
import jax.numpy as jnp
from jax.experimental import pallas as pl
from jax.experimental.pallas import tpu as pltpu

_E = 8
_TOP_P = 0.7
_EPS = 1e-10


def _router_block(x_ref, wg_ref, w_ref, dec_ref, loss_ref, imp_acc, ent_acc):
    i = pl.program_id(0)
    nsteps = pl.num_programs(0)

    @pl.when(i == 0)
    def _init():
        imp_acc[...] = jnp.zeros_like(imp_acc)
        ent_acc[...] = jnp.zeros_like(ent_acc)

    x = x_ref[...]
    wg = wg_ref[...]
    logits = jax.lax.dot_general(
        x, wg, (((1,), (1,)), ((), ())),
        preferred_element_type=jnp.float32)
    lt = logits.T

    m = jnp.max(lt, axis=0, keepdims=True)
    ex = jnp.exp(lt - m)
    p = ex / jnp.sum(ex, axis=0, keepdims=True)

    row = jax.lax.broadcasted_iota(jnp.int32, p.shape, 0)
    s_rows = []
    r_rows = []
    for e in range(_E):
        pe = p[e:e + 1, :]
        higher = (p > pe) | ((p == pe) & (row < e))
        s_rows.append(jnp.sum(jnp.where(higher, p, 0.0), axis=0, keepdims=True))
        r_rows.append(jnp.sum(higher.astype(jnp.int32), axis=0, keepdims=True))
    s_above = jnp.concatenate(s_rows, axis=0)
    rank = jnp.concatenate(r_rows, axis=0)

    kept = s_above <= _TOP_P
    w_ref[...] = kept.astype(jnp.int8)
    cnt = jnp.sum(kept.astype(jnp.int32), axis=0, keepdims=True)
    dec_ref[...] = (cnt > 1).astype(jnp.int32)

    contrib = jnp.where(kept, p, 0.0)
    imp_rows = [
        jnp.sum(jnp.where(rank == k, contrib, 0.0), axis=0, keepdims=True)
        for k in range(_E)
    ]
    imp_acc[...] += jnp.concatenate(imp_rows, axis=0)
    ent_acc[...] += p * jnp.log(p + _EPS)

    @pl.when(i == nsteps - 1)
    def _fin():
        imp = jnp.sum(imp_acc[...], axis=1)
        mean = jnp.mean(imp)
        var = jnp.sum((imp - mean) ** 2) / (_E - 1)
        loss_imp = var / (mean * mean + _EPS)
        n_tokens = nsteps * x_ref.shape[0]
        loss_dyn = -jnp.sum(ent_acc[...]) / n_tokens
        loss_ref[0, 0] = loss_imp + 0.1 * loss_dyn


@functools.partial(jax.jit, static_argnames=())
def kernel(x, W_gate, W_noise):
    del W_noise
    b, n, d = x.shape
    e = W_gate.shape[0]
    bn = b * n
    t = 512
    grid = bn // t
    x_flat = x.reshape(bn, d)

    w_i8, dec, loss = pl.pallas_call(
        _router_block,
        grid=(grid,),
        in_specs=[
            pl.BlockSpec((t, d), lambda i: (i, 0)),
            pl.BlockSpec((e, d), lambda i: (0, 0)),
        ],
        out_specs=[
            pl.BlockSpec((e, t), lambda i: (0, i)),
            pl.BlockSpec((1, t), lambda i: (0, i)),
            pl.BlockSpec(memory_space=pltpu.SMEM),
        ],
        out_shape=[
            jax.ShapeDtypeStruct((e, bn), jnp.int8),
            jax.ShapeDtypeStruct((1, bn), jnp.int32),
            jax.ShapeDtypeStruct((1, 1), jnp.float32),
        ],
        scratch_shapes=[
            pltpu.VMEM((e, t), jnp.float32),
            pltpu.VMEM((e, t), jnp.float32),
        ],
        compiler_params=pltpu.CompilerParams(
            dimension_semantics=("arbitrary",),
        ),
    )(x_flat, W_gate)

    expert_weights = w_i8.T.astype(jnp.bool_).reshape(b, n, e)
    expert_decisions = dec.reshape(b, n)
    gating_loss = loss.reshape(())
    return expert_weights, expert_decisions, gating_loss


# speedup vs baseline: 12.3085x; 1.1273x over previous
import functools

import jax
import jax.numpy as jnp
from jax.experimental import pallas as pl
from jax.experimental.pallas import tpu as pltpu

_E = 8
_TOP_P = 0.7
_EPS = 1e-10


def _router_block(x_ref, wg_ref, w_ref, dec_ref, loss_ref, imp_acc, ent_acc):
    i = pl.program_id(0)
    nsteps = pl.num_programs(0)

    @pl.when(i == 0)
    def _init():
        imp_acc[...] = jnp.zeros_like(imp_acc)
        ent_acc[...] = jnp.zeros_like(ent_acc)

    x = x_ref[...]
    wg = wg_ref[...]
    logits = jax.lax.dot_general(
        x, wg, (((1,), (1,)), ((), ())),
        preferred_element_type=jnp.float32)
    lt = logits.T

    m = jnp.max(lt, axis=0, keepdims=True)
    ex = jnp.exp(lt - m)
    p = ex / jnp.sum(ex, axis=0, keepdims=True)

    row = jax.lax.broadcasted_iota(jnp.int32, p.shape, 0)
    s_rows = []
    r_rows = []
    for e in range(_E):
        pe = p[e:e + 1, :]
        higher = (p > pe) | ((p == pe) & (row < e))
        s_rows.append(jnp.sum(jnp.where(higher, p, 0.0), axis=0, keepdims=True))
        r_rows.append(jnp.sum(higher.astype(jnp.int32), axis=0, keepdims=True))
    s_above = jnp.concatenate(s_rows, axis=0)
    rank = jnp.concatenate(r_rows, axis=0)

    kept = s_above <= _TOP_P
    w_ref[...] = kept.astype(jnp.int8)
    cnt = jnp.sum(kept.astype(jnp.int32), axis=0, keepdims=True)
    dec_ref[...] = (cnt > 1).astype(jnp.int32)

    contrib = jnp.where(kept, p, 0.0)
    imp_rows = [
        jnp.sum(jnp.where(rank == k, contrib, 0.0), axis=0, keepdims=True)
        for k in range(_E)
    ]
    imp_acc[...] += jnp.concatenate(imp_rows, axis=0)
    ent_acc[...] += p * jnp.log(p + _EPS)

    @pl.when(i == nsteps - 1)
    def _fin():
        imp = jnp.sum(imp_acc[...], axis=1)
        mean = jnp.mean(imp)
        var = jnp.sum((imp - mean) ** 2) / (_E - 1)
        loss_imp = var / (mean * mean + _EPS)
        n_tokens = nsteps * x_ref.shape[0]
        loss_dyn = -jnp.sum(ent_acc[...]) / n_tokens
        loss_ref[0, 0] = loss_imp + 0.1 * loss_dyn


@functools.partial(jax.jit, static_argnames=())
def kernel(x, W_gate, W_noise):
    del W_noise
    b, n, d = x.shape
    e = W_gate.shape[0]
    bn = b * n
    t = 1024
    grid = bn // t
    x_flat = x.reshape(bn, d)

    w_i8, dec, loss = pl.pallas_call(
        _router_block,
        grid=(grid,),
        in_specs=[
            pl.BlockSpec((t, d), lambda i: (i, 0)),
            pl.BlockSpec((e, d), lambda i: (0, 0)),
        ],
        out_specs=[
            pl.BlockSpec((e, t), lambda i: (0, i)),
            pl.BlockSpec((1, t), lambda i: (0, i)),
            pl.BlockSpec(memory_space=pltpu.SMEM),
        ],
        out_shape=[
            jax.ShapeDtypeStruct((e, bn), jnp.int8),
            jax.ShapeDtypeStruct((1, bn), jnp.int32),
            jax.ShapeDtypeStruct((1, 1), jnp.float32),
        ],
        scratch_shapes=[
            pltpu.VMEM((e, t), jnp.float32),
            pltpu.VMEM((e, t), jnp.float32),
        ],
        compiler_params=pltpu.CompilerParams(
            dimension_semantics=("arbitrary",),
        ),
    )(x_flat, W_gate)

    expert_weights = w_i8.T.astype(jnp.bool_).reshape(b, n, e)
    expert_decisions = dec.reshape(b, n)
    gating_loss = loss.reshape(())
    return expert_weights, expert_decisions, gating_loss
